# baseline (speedup 1.0000x reference)
import jax
import jax.numpy as jnp
from jax import lax
from jax.experimental import pallas as pl
from jax.experimental.pallas import tpu as pltpu
from jax.experimental.pallas import tpu_sc as plsc

B, PNUM, G, P = 8, 100, 16, 128
D = P * 2
ROWS_PER_W = 25
BATCH_PER_CORE = 4
LN2 = 0.6931471805599453
_LOG2C = (-2.5123769e-02, 1.1930088e-01, -2.7462757e-01, 4.5553029e-01,
          -7.1755898e-01, 1.4424754e+00, 2.1172477e-06)


def _ln_vec(x):
    bits = lax.bitcast_convert_type(x, jnp.int32)
    e = lax.shift_right_arithmetic(bits, jnp.full((16,), 23, jnp.int32))
    e = e - jnp.full((16,), 127, jnp.int32)
    mbits = lax.bitwise_or(
        lax.bitwise_and(bits, jnp.full((16,), 0x7FFFFF, jnp.int32)),
        jnp.full((16,), 0x3F800000, jnp.int32))
    t = lax.bitcast_convert_type(mbits, jnp.float32) - 1.0
    p = jnp.full((16,), _LOG2C[0], jnp.float32)
    for c in _LOG2C[1:]:
        p = p * t + c
    return (e.astype(jnp.float32) + p) * LN2


def _sc_body(a_hbm, g_hbm, pc_hbm, out_hbm,
             a_v, gn_v, g_v, chunk_v, cost_v, pc_v, vec_v, part_v,
             sh_cost, sh_part):
    c = lax.axis_index("c")
    s = lax.axis_index("s")
    lb = s // BATCH_PER_CORE
    b = c * BATCH_PER_CORE + lb
    r = s % BATCH_PER_CORE
    r0 = r * ROWS_PER_W
    iota = lax.iota(jnp.int32, 16)

    pltpu.sync_copy(a_hbm.at[pl.ds((b * PNUM + r0) * D, ROWS_PER_W * D)], a_v)
    pltpu.sync_copy(g_hbm.at[pl.ds(b * G * D, G * D)], gn_v)

    jmul = iota * D

    def t_step(e, _):
        g_v[e] = plsc.load_gather(gn_v, [jmul + e])
        return 0
    lax.fori_loop(0, D, t_step, 0)

    def g2_step(e, acc):
        ge = g_v[e]
        return acc + ge * ge
    sum_g2 = lax.fori_loop(0, D, g2_step, jnp.zeros((16,), jnp.float32),
                           unroll=8)

    def row_step(i, _):
        base = i * D

        def a2_step(k, acc):
            va = a_v[pl.ds(base + k * 16, 16)]
            return acc + va * va
        sa2 = jnp.sum(lax.fori_loop(0, D // 16, a2_step,
                                    jnp.zeros((16,), jnp.float32), unroll=4))

        def cross_step(k, acc):
            va = a_v[pl.ds(base + k * 16, 16)]
            for l in range(16):
                acc = acc + va[l] * g_v[k * 16 + l]
            return acc
        cross = lax.fori_loop(0, D // 16, cross_step,
                              jnp.zeros((16,), jnp.float32))
        chunk_v[i] = (sa2 + sum_g2 - 2.0 * cross) * (1.0 / D)
        return 0
    lax.fori_loop(0, ROWS_PER_W, row_step, 0)

    pltpu.sync_copy(chunk_v, sh_cost.at[lb, pl.ds(r0, ROWS_PER_W)])
    plsc.subcore_barrier()

    @pl.when(r == 0)
    def _leader():
        pltpu.sync_copy(sh_cost.at[lb], cost_v)
        pltpu.sync_copy(pc_hbm, pc_v)
        poff = b * PNUM
        size_vec = plsc.load_gather(pc_v, [jnp.full((16,), B * PNUM, jnp.int32)
                                           + b])
        m_m = (PNUM / size_vec)[0]
        m_u = (PNUM / (PNUM - size_vec))[0]

        def match_step(k, carry):
            h_acc, src_vec, colmask = carry

            def scan_row(i, bc):
                bestv, bestr = bc
                ci = cost_v[i] + colmask
                lt = ci < bestv
                bestv = jnp.where(lt, ci, bestv)
                bestr = jnp.where(lt, jnp.full((16,), i, jnp.int32), bestr)
                return bestv, bestr
            bestv, bestr = lax.fori_loop(
                0, PNUM, scan_row,
                (jnp.full((16,), 1e9, jnp.float32),
                 jnp.zeros((16,), jnp.int32)), unroll=4)

            vmin = jnp.min(bestv)
            jv = plsc.all_reduce_ffs(bestv == vmin)
            lane_j = iota == jv
            i_star = jnp.max(jnp.where(lane_j, bestr,
                                       jnp.full((16,), -1, jnp.int32)))
            src_vec = jnp.where(iota == k, jnp.full((16,), i_star, jnp.int32),
                                src_vec)
            colmask = jnp.where(lane_j, jnp.full((16,), 1e9, jnp.float32),
                                colmask)
            cost_v[i_star] = jnp.full((16,), 1e9, jnp.float32)
            return h_acc + vmin, src_vec, colmask

        h_sum, src_vec, _ = lax.fori_loop(
            0, G, match_step,
            (jnp.float32(0.0), jnp.zeros((16,), jnp.int32),
             jnp.zeros((16,), jnp.float32)))

        def l_step(k, acc):
            pv = pc_v[pl.ds(poff + k * 16, 16)]
            return acc + _ln_vec(1.0 - pv)
        acc_l = lax.fori_loop(0, 6, l_step, jnp.zeros((16,), jnp.float32))
        pv_t = pc_v[pl.ds(poff + 84, 16)]
        acc_l = acc_l + jnp.where(iota >= 12, _ln_vec(1.0 - pv_t),
                                  jnp.zeros((16,), jnp.float32))
        sum_l = jnp.sum(acc_l)

        p_src = plsc.load_gather(pc_v, [src_vec + poff])
        sum_l_src = jnp.sum(_ln_vec(1.0 - p_src))
        pp = jnp.maximum(p_src, 0.05)
        sum_term = jnp.sum(1.0 / pp - 1.0)

        loss_b = m_u * (sum_l_src - sum_l) + m_m * sum_term
        partial = loss_b * (1.0 / (B * PNUM)) + h_sum * (0.1 / (B * G))
        vec_v[...] = jnp.where(iota == 0, jnp.full((16,), partial),
                               jnp.zeros((16,), jnp.float32))
        pltpu.sync_copy(vec_v, sh_part.at[lb])

    plsc.subcore_barrier()

    @pl.when(s == 0)
    def _core_leader():
        pltpu.sync_copy(sh_part, part_v)
        acc = part_v[0] + part_v[1] + part_v[2] + part_v[3]
        total = jnp.sum(acc)
        vec_v[...] = jnp.full((16,), total)
        pltpu.sync_copy(vec_v, out_hbm.at[c])


@jax.jit
def _run(a1d, g1d, pc):
    mesh = plsc.VectorSubcoreMesh(core_axis_name="c", subcore_axis_name="s")
    f = pl.kernel(
        _sc_body,
        out_type=jax.ShapeDtypeStruct((2, 16), jnp.float32),
        mesh=mesh,
        scratch_types=[
            pltpu.VMEM((ROWS_PER_W * D,), jnp.float32),
            pltpu.VMEM((G * D,), jnp.float32),
            pltpu.VMEM((D, 16), jnp.float32),
            pltpu.VMEM((ROWS_PER_W, 16), jnp.float32),
            pltpu.VMEM((PNUM, 16), jnp.float32),
            pltpu.VMEM((B * PNUM + 16,), jnp.float32),
            pltpu.VMEM((16,), jnp.float32),
            pltpu.VMEM((BATCH_PER_CORE, 16), jnp.float32),
            pltpu.VMEM_SHARED((BATCH_PER_CORE, PNUM, 16), jnp.float32),
            pltpu.VMEM_SHARED((BATCH_PER_CORE, 16), jnp.float32),
        ],
        compiler_params=pltpu.CompilerParams(use_tc_tiling_on_sc=False,
                                             needs_layout_passes=False),
    )
    return f(a1d, g1d, pc)


def kernel(pred_poly, pred_prob, gt_py, gt_num):
    a1d = pred_poly.reshape(-1)
    g1d = gt_py.reshape(-1)
    pc = jnp.concatenate([pred_prob.reshape(-1),
                          gt_num.astype(jnp.float32),
                          jnp.zeros((8,), jnp.float32)])
    out = _run(a1d, g1d, pc)
    return out[0, 0] + out[1, 0]


# speedup vs baseline: 2.6368x; 2.6368x over previous
import jax
import jax.numpy as jnp
from jax import lax
from jax.experimental import pallas as pl
from jax.experimental.pallas import tpu as pltpu
from jax.experimental.pallas import tpu_sc as plsc

B, PNUM, G, P = 8, 100, 16, 128
D = P * 2
ROWS_PER_W = 25
BATCH_PER_CORE = 4
LN2 = 0.6931471805599453
_LOG2C = (-2.5123769e-02, 1.1930088e-01, -2.7462757e-01, 4.5553029e-01,
          -7.1755898e-01, 1.4424754e+00, 2.1172477e-06)


def _ln_vec(x):
    bits = lax.bitcast_convert_type(x, jnp.int32)
    e = lax.shift_right_arithmetic(bits, jnp.full((16,), 23, jnp.int32))
    e = e - jnp.full((16,), 127, jnp.int32)
    mbits = lax.bitwise_or(
        lax.bitwise_and(bits, jnp.full((16,), 0x7FFFFF, jnp.int32)),
        jnp.full((16,), 0x3F800000, jnp.int32))
    t = lax.bitcast_convert_type(mbits, jnp.float32) - 1.0
    p = jnp.full((16,), _LOG2C[0], jnp.float32)
    for c in _LOG2C[1:]:
        p = p * t + c
    return (e.astype(jnp.float32) + p) * LN2


def _sc_body(a_hbm, p_hbm, g_hbm, n_hbm, out_hbm,
             a_v, gn_v, g_v, chunk_v, cost_v, pc_v, sz_v, vec_v, part_v,
             sh_cost, sh_part):
    c = lax.axis_index("c")
    s = lax.axis_index("s")
    lb = s // BATCH_PER_CORE
    b = c * BATCH_PER_CORE + lb
    r = s % BATCH_PER_CORE
    r0 = r * ROWS_PER_W
    iota = lax.iota(jnp.int32, 16)
    pltpu.sync_copy(a_hbm.at[b, pl.ds(r0, ROWS_PER_W)], a_v)
    pltpu.sync_copy(g_hbm.at[b], gn_v)

    def t_step(e, _):
        g_v[e] = plsc.load_gather(gn_v, [iota, jnp.full((16,), e, jnp.int32)])
        return 0
    lax.fori_loop(0, D, t_step, 0)

    def g2_step(e, acc):
        ge = g_v[e]
        return acc + ge * ge
    sum_g2 = lax.fori_loop(0, D, g2_step, jnp.zeros((16,), jnp.float32),
                           unroll=8)

    def row_step(i, _):
        def a2_step(k, acc):
            va = a_v[i, pl.ds(k * 16, 16)]
            return acc + va * va
        sa2 = jnp.sum(lax.fori_loop(0, D // 16, a2_step,
                                    jnp.zeros((16,), jnp.float32), unroll=4))

        def cross_step(k, acc):
            va = a_v[i, pl.ds(k * 16, 16)]
            for l in range(16):
                acc = acc + va[l] * g_v[k * 16 + l]
            return acc
        cross = lax.fori_loop(0, D // 16, cross_step,
                              jnp.zeros((16,), jnp.float32))
        chunk_v[i] = (sa2 + sum_g2 - 2.0 * cross) * (1.0 / D)
        return 0
    lax.fori_loop(0, ROWS_PER_W, row_step, 0)

    pltpu.sync_copy(chunk_v, sh_cost.at[lb, pl.ds(r0, ROWS_PER_W)])
    plsc.subcore_barrier()

    @pl.when(r == 0)
    def _leader():
        pltpu.sync_copy(sh_cost.at[lb], cost_v)
        pltpu.sync_copy(p_hbm, pc_v)
        pltpu.sync_copy(n_hbm, sz_v)
        size_vec = plsc.load_gather(sz_v, [jnp.full((16,), 0, jnp.int32) + b]
                                    ).astype(jnp.float32)
        m_m = (PNUM / size_vec)[0]
        m_u = (PNUM / (PNUM - size_vec))[0]

        def match_step(k, carry):
            h_acc, src_vec, colmask = carry

            def scan_row(i, bc):
                bestv, bestr = bc
                ci = cost_v[i] + colmask
                lt = ci < bestv
                bestv = jnp.where(lt, ci, bestv)
                bestr = jnp.where(lt, jnp.full((16,), i, jnp.int32), bestr)
                return bestv, bestr
            bestv, bestr = lax.fori_loop(
                0, PNUM, scan_row,
                (jnp.full((16,), 1e9, jnp.float32),
                 jnp.zeros((16,), jnp.int32)), unroll=4)

            vmin = jnp.min(bestv)
            jv = plsc.all_reduce_ffs(bestv == vmin)
            lane_j = iota == jv
            i_star = jnp.max(jnp.where(lane_j, bestr,
                                       jnp.full((16,), -1, jnp.int32)))
            src_vec = jnp.where(iota == k, jnp.full((16,), i_star, jnp.int32),
                                src_vec)
            colmask = jnp.where(lane_j, jnp.full((16,), 1e9, jnp.float32),
                                colmask)
            cost_v[i_star] = jnp.full((16,), 1e9, jnp.float32)
            return h_acc + vmin, src_vec, colmask

        h_sum, src_vec, _ = lax.fori_loop(
            0, G, match_step,
            (jnp.float32(0.0), jnp.zeros((16,), jnp.int32),
             jnp.zeros((16,), jnp.float32)))

        def l_step(k, acc):
            pv = pc_v[b, pl.ds(k * 16, 16)]
            return acc + _ln_vec(1.0 - pv)
        acc_l = lax.fori_loop(0, 6, l_step, jnp.zeros((16,), jnp.float32))
        pv_t = pc_v[b, pl.ds(84, 16)]
        acc_l = acc_l + jnp.where(iota >= 12, _ln_vec(1.0 - pv_t),
                                  jnp.zeros((16,), jnp.float32))
        sum_l = jnp.sum(acc_l)

        p_src = plsc.load_gather(pc_v, [jnp.full((16,), b, jnp.int32),
                                        src_vec])
        sum_l_src = jnp.sum(_ln_vec(1.0 - p_src))
        pp = jnp.maximum(p_src, 0.05)
        sum_term = jnp.sum(1.0 / pp - 1.0)

        loss_b = m_u * (sum_l_src - sum_l) + m_m * sum_term
        partial = loss_b * (1.0 / (B * PNUM)) + h_sum * (0.1 / (B * G))
        vec_v[...] = jnp.where(iota == 0, jnp.full((16,), partial),
                               jnp.zeros((16,), jnp.float32))
        pltpu.sync_copy(vec_v, sh_part.at[lb])

    plsc.subcore_barrier()

    @pl.when(s == 0)
    def _core_leader():
        pltpu.sync_copy(sh_part, part_v)
        acc = part_v[0] + part_v[1] + part_v[2] + part_v[3]
        total = jnp.sum(acc)
        vec_v[...] = jnp.full((16,), total)
        pltpu.sync_copy(vec_v, out_hbm.at[c])


@jax.jit
def _run(pred_poly, pred_prob, gt_py, gt_num):
    mesh = plsc.VectorSubcoreMesh(core_axis_name="c", subcore_axis_name="s")
    f = pl.kernel(
        _sc_body,
        out_type=jax.ShapeDtypeStruct((2, 16), jnp.float32),
        mesh=mesh,
        scratch_types=[
            pltpu.VMEM((ROWS_PER_W, D), jnp.float32),
            pltpu.VMEM((G, D), jnp.float32),
            pltpu.VMEM((D, 16), jnp.float32),
            pltpu.VMEM((ROWS_PER_W, 16), jnp.float32),
            pltpu.VMEM((PNUM, 16), jnp.float32),
            pltpu.VMEM((B, PNUM), jnp.float32),
            pltpu.VMEM((B,), jnp.int32),
            pltpu.VMEM((16,), jnp.float32),
            pltpu.VMEM((BATCH_PER_CORE, 16), jnp.float32),
            pltpu.VMEM_SHARED((BATCH_PER_CORE, PNUM, 16), jnp.float32),
            pltpu.VMEM_SHARED((BATCH_PER_CORE, 16), jnp.float32),
        ],
        compiler_params=pltpu.CompilerParams(use_tc_tiling_on_sc=False,
                                             needs_layout_passes=False),
    )
    return f(pred_poly, pred_prob, gt_py, gt_num)


def kernel(pred_poly, pred_prob, gt_py, gt_num):
    a = pred_poly.reshape(B, PNUM, D)
    g = gt_py.reshape(B, G, D)
    out = _run(a, pred_prob, g, gt_num)
    return out[0, 0] + out[1, 0]


# speedup vs baseline: 3.1452x; 1.1928x over previous
import jax
import jax.numpy as jnp
from jax import lax
from jax.experimental import pallas as pl
from jax.experimental.pallas import tpu as pltpu
from jax.experimental.pallas import tpu_sc as plsc

B, PNUM, G, P = 8, 100, 16, 128
D = P * 2
ROWS_PER_W = 25
BATCH_PER_CORE = 4
LN2 = 0.6931471805599453
_LOG2C = (-2.5123769e-02, 1.1930088e-01, -2.7462757e-01, 4.5553029e-01,
          -7.1755898e-01, 1.4424754e+00, 2.1172477e-06)


def _ln_vec(x):
    bits = lax.bitcast_convert_type(x, jnp.int32)
    e = lax.shift_right_arithmetic(bits, jnp.full((16,), 23, jnp.int32))
    e = e - jnp.full((16,), 127, jnp.int32)
    mbits = lax.bitwise_or(
        lax.bitwise_and(bits, jnp.full((16,), 0x7FFFFF, jnp.int32)),
        jnp.full((16,), 0x3F800000, jnp.int32))
    t = lax.bitcast_convert_type(mbits, jnp.float32) - 1.0
    p = jnp.full((16,), _LOG2C[0], jnp.float32)
    for c in _LOG2C[1:]:
        p = p * t + c
    return (e.astype(jnp.float32) + p) * LN2


def _sc_body(a_hbm, p_hbm, g_hbm, n_hbm, out_hbm,
             a_v, gn_v, chunk_v, cost_v, pc_v, sz_v, vec_v, part_v,
             sh_cost, sh_part):
    c = lax.axis_index("c")
    s = lax.axis_index("s")
    lb = s // BATCH_PER_CORE
    b = c * BATCH_PER_CORE + lb
    r = s % BATCH_PER_CORE
    r0 = r * ROWS_PER_W
    iota = lax.iota(jnp.int32, 16)
    pltpu.sync_copy(a_hbm.at[b, pl.ds(r0, ROWS_PER_W)], a_v)
    pltpu.sync_copy(g_hbm.at[b], gn_v)

    NC = D // 16
    zero16 = jnp.zeros((16,), jnp.float32)

    sg2_vec = zero16
    for j in range(G):
        acc = zero16
        for cch in range(NC):
            gc = gn_v[j, pl.ds(cch * 16, 16)]
            acc = acc + gc * gc
        sg2_vec = jnp.where(iota == j, jnp.full((16,), jnp.sum(acc)), sg2_vec)

    def pair_step(t, _):
        i0 = t * 2
        i1 = i0 + 1
        av0 = [a_v[i0, pl.ds(cch * 16, 16)] for cch in range(NC)]
        av1 = [a_v[i1, pl.ds(cch * 16, 16)] for cch in range(NC)]
        acc0 = zero16
        acc1 = zero16
        for cch in range(NC):
            acc0 = acc0 + av0[cch] * av0[cch]
            acc1 = acc1 + av1[cch] * av1[cch]
        sa2_0 = jnp.sum(acc0)
        sa2_1 = jnp.sum(acc1)
        cr0 = zero16
        cr1 = zero16
        for j in range(G):
            gc = gn_v[j, pl.ds(0, 16)]
            c0 = av0[0] * gc
            c1 = av1[0] * gc
            for cch in range(1, NC):
                gc = gn_v[j, pl.ds(cch * 16, 16)]
                c0 = c0 + av0[cch] * gc
                c1 = c1 + av1[cch] * gc
            lane = iota == j
            cr0 = jnp.where(lane, jnp.full((16,), jnp.sum(c0)), cr0)
            cr1 = jnp.where(lane, jnp.full((16,), jnp.sum(c1)), cr1)
        chunk_v[i0] = (sa2_0 + sg2_vec - 2.0 * cr0) * (1.0 / D)
        chunk_v[i1] = (sa2_1 + sg2_vec - 2.0 * cr1) * (1.0 / D)
        return 0
    lax.fori_loop(0, ROWS_PER_W // 2, pair_step, 0)

    i_t = ROWS_PER_W - 1
    av_t = [a_v[i_t, pl.ds(cch * 16, 16)] for cch in range(NC)]
    acc_t = zero16
    for cch in range(NC):
        acc_t = acc_t + av_t[cch] * av_t[cch]
    sa2_t = jnp.sum(acc_t)
    cr_t = zero16
    for j in range(G):
        ct = av_t[0] * gn_v[j, pl.ds(0, 16)]
        for cch in range(1, NC):
            ct = ct + av_t[cch] * gn_v[j, pl.ds(cch * 16, 16)]
        cr_t = jnp.where(iota == j, jnp.full((16,), jnp.sum(ct)), cr_t)
    chunk_v[i_t] = (sa2_t + sg2_vec - 2.0 * cr_t) * (1.0 / D)

    pltpu.sync_copy(chunk_v, sh_cost.at[lb, pl.ds(r0, ROWS_PER_W)])
    plsc.subcore_barrier()

    @pl.when(r == 0)
    def _leader():
        pltpu.sync_copy(sh_cost.at[lb], cost_v)
        pltpu.sync_copy(p_hbm, pc_v)
        pltpu.sync_copy(n_hbm, sz_v)
        size_vec = plsc.load_gather(sz_v, [jnp.full((16,), 0, jnp.int32) + b]
                                    ).astype(jnp.float32)
        m_m = (PNUM / size_vec)[0]
        m_u = (PNUM / (PNUM - size_vec))[0]

        def match_step(k, carry):
            h_acc, src_vec, colmask = carry

            def scan_row(i, bc):
                bestv, bestr = bc
                ci = cost_v[i] + colmask
                lt = ci < bestv
                bestv = jnp.where(lt, ci, bestv)
                bestr = jnp.where(lt, jnp.full((16,), i, jnp.int32), bestr)
                return bestv, bestr
            bestv, bestr = lax.fori_loop(
                0, PNUM, scan_row,
                (jnp.full((16,), 1e9, jnp.float32),
                 jnp.zeros((16,), jnp.int32)), unroll=4)

            vmin = jnp.min(bestv)
            jv = plsc.all_reduce_ffs(bestv == vmin)
            lane_j = iota == jv
            i_star = jnp.max(jnp.where(lane_j, bestr,
                                       jnp.full((16,), -1, jnp.int32)))
            src_vec = jnp.where(iota == k, jnp.full((16,), i_star, jnp.int32),
                                src_vec)
            colmask = jnp.where(lane_j, jnp.full((16,), 1e9, jnp.float32),
                                colmask)
            cost_v[i_star] = jnp.full((16,), 1e9, jnp.float32)
            return h_acc + vmin, src_vec, colmask

        h_sum, src_vec, _ = lax.fori_loop(
            0, G, match_step,
            (jnp.float32(0.0), jnp.zeros((16,), jnp.int32),
             jnp.zeros((16,), jnp.float32)))

        def l_step(k, acc):
            pv = pc_v[b, pl.ds(k * 16, 16)]
            return acc + _ln_vec(1.0 - pv)
        acc_l = lax.fori_loop(0, 6, l_step, jnp.zeros((16,), jnp.float32))
        pv_t = pc_v[b, pl.ds(84, 16)]
        acc_l = acc_l + jnp.where(iota >= 12, _ln_vec(1.0 - pv_t),
                                  jnp.zeros((16,), jnp.float32))
        sum_l = jnp.sum(acc_l)

        p_src = plsc.load_gather(pc_v, [jnp.full((16,), b, jnp.int32),
                                        src_vec])
        sum_l_src = jnp.sum(_ln_vec(1.0 - p_src))
        pp = jnp.maximum(p_src, 0.05)
        sum_term = jnp.sum(1.0 / pp - 1.0)

        loss_b = m_u * (sum_l_src - sum_l) + m_m * sum_term
        partial = loss_b * (1.0 / (B * PNUM)) + h_sum * (0.1 / (B * G))
        vec_v[...] = jnp.where(iota == 0, jnp.full((16,), partial),
                               jnp.zeros((16,), jnp.float32))
        pltpu.sync_copy(vec_v, sh_part.at[lb])

    plsc.subcore_barrier()

    @pl.when(s == 0)
    def _core_leader():
        pltpu.sync_copy(sh_part, part_v)
        acc = part_v[0] + part_v[1] + part_v[2] + part_v[3]
        total = jnp.sum(acc)
        vec_v[...] = jnp.full((16,), total)
        pltpu.sync_copy(vec_v, out_hbm.at[c])


@jax.jit
def _run(pred_poly, pred_prob, gt_py, gt_num):
    mesh = plsc.VectorSubcoreMesh(core_axis_name="c", subcore_axis_name="s")
    f = pl.kernel(
        _sc_body,
        out_type=jax.ShapeDtypeStruct((2, 16), jnp.float32),
        mesh=mesh,
        scratch_types=[
            pltpu.VMEM((ROWS_PER_W, D), jnp.float32),
            pltpu.VMEM((G, D), jnp.float32),
            pltpu.VMEM((ROWS_PER_W, 16), jnp.float32),
            pltpu.VMEM((PNUM, 16), jnp.float32),
            pltpu.VMEM((B, PNUM), jnp.float32),
            pltpu.VMEM((B,), jnp.int32),
            pltpu.VMEM((16,), jnp.float32),
            pltpu.VMEM((BATCH_PER_CORE, 16), jnp.float32),
            pltpu.VMEM_SHARED((BATCH_PER_CORE, PNUM, 16), jnp.float32),
            pltpu.VMEM_SHARED((BATCH_PER_CORE, 16), jnp.float32),
        ],
        compiler_params=pltpu.CompilerParams(use_tc_tiling_on_sc=False,
                                             needs_layout_passes=False),
    )
    return f(pred_poly, pred_prob, gt_py, gt_num)


def kernel(pred_poly, pred_prob, gt_py, gt_num):
    a = pred_poly.reshape(B, PNUM, D)
    g = gt_py.reshape(B, G, D)
    out = _run(a, pred_prob, g, gt_num)
    return out[0, 0] + out[1, 0]
